# baseline (speedup 1.0000x reference)
import functools

import jax
import jax.numpy as jnp
from jax import lax
from jax.experimental import pallas as pl
from jax.experimental.pallas import tpu as pltpu
from jax.experimental.pallas import tpu_sc as plsc

_LANES = 16
_CHUNK = 128


def _build(B, S, V, D, NC, NS):
    N = B * S
    mesh = plsc.VectorSubcoreMesh(core_axis_name="c", subcore_axis_name="s")

    @functools.partial(
        pl.kernel,
        mesh=mesh,
        out_type=jax.ShapeDtypeStruct((N, D), jnp.float32),
        scratch_types=[
            pltpu.VMEM((B, _CHUNK), jnp.int32),
            pltpu.VMEM((_CHUNK, D), jnp.float32),
            pltpu.VMEM((B, _CHUNK, D), jnp.float32),
            pltpu.SemaphoreType.DMA,
            pltpu.SemaphoreType.DMA,
            pltpu.SemaphoreType.DMA,
            pltpu.SemaphoreType.DMA,
        ],
    )
    def emb(ids_hbm, tok_hbm, pos_hbm, out_hbm, idx_v, pos_v, tok_v, sem_p, sem_i, sem_g, sem_s):
        wid = lax.axis_index("s") * NC + lax.axis_index("c")
        off = wid * _CHUNK
        idx_cps = [
            pltpu.async_copy(ids_hbm.at[b, pl.ds(off, _CHUNK)], idx_v.at[b], sem_i)
            for b in range(B)
        ]
        pos_cp = pltpu.async_copy(pos_hbm.at[pl.ds(off, _CHUNK)], pos_v, sem_p)
        gathers = []
        for b in range(B):
            idx_cps[b].wait()
            gathers.append(
                pltpu.async_copy(tok_hbm.at[idx_v.at[b]], tok_v.at[b], sem_g)
            )
        for b in range(B):
            gathers[b].wait()
        pos_cp.wait()
        qrows = _CHUNK // 2
        stores = []
        for q in range(2):

            @plsc.parallel_loop(q * qrows, (q + 1) * qrows, step=1, unroll=4)
            def add_row(r):
                for c in range(D // _LANES):
                    v = pos_v[r, pl.ds(c * _LANES, _LANES)]
                    for b in range(B):
                        plsc.addupdate(tok_v.at[b, r, pl.ds(c * _LANES, _LANES)], v)

            for b in range(B):
                stores.append(
                    pltpu.async_copy(
                        tok_v.at[b, pl.ds(q * qrows, qrows)],
                        out_hbm.at[pl.ds(b * S + off + q * qrows, qrows)],
                        sem_s,
                    )
                )
        for cp in stores:
            cp.wait()

    return emb


def kernel(input_ids, token_weight, pos_weight):
    B, S = input_ids.shape
    V, D = token_weight.shape
    info = plsc.get_sparse_core_info()
    NC, NS = info.num_cores, info.num_subcores
    emb = _build(B, S, V, D, NC, NS)
    out = emb(input_ids.astype(jnp.int32), token_weight, pos_weight)
    return out.reshape(B, S, D)


# speedup vs baseline: 1.0395x; 1.0395x over previous
import functools

import jax
import jax.numpy as jnp
from jax import lax
from jax.experimental import pallas as pl
from jax.experimental.pallas import tpu as pltpu
from jax.experimental.pallas import tpu_sc as plsc

_LANES = 16
_CHUNK = 128


def _build(B, S, V, D, NC, NS):
    N = B * S
    mesh = plsc.VectorSubcoreMesh(core_axis_name="c", subcore_axis_name="s")

    @functools.partial(
        pl.kernel,
        mesh=mesh,
        out_type=jax.ShapeDtypeStruct((N, D), jnp.float32),
        scratch_types=[
            pltpu.VMEM((B, _CHUNK), jnp.int32),
            pltpu.VMEM((_CHUNK, D), jnp.float32),
            pltpu.VMEM((B, _CHUNK, D), jnp.float32),
            pltpu.SemaphoreType.DMA,
            pltpu.SemaphoreType.DMA,
            pltpu.SemaphoreType.DMA,
            pltpu.SemaphoreType.DMA,
        ],
    )
    def emb(ids_hbm, tok_hbm, pos_hbm, out_hbm, idx_v, pos_v, tok_v, sem_p, sem_i, sem_g, sem_s):
        wid = lax.axis_index("s") * NC + lax.axis_index("c")
        off = wid * _CHUNK
        idx_cps = [
            pltpu.async_copy(ids_hbm.at[b, pl.ds(off, _CHUNK)], idx_v.at[b], sem_i)
            for b in range(B)
        ]
        pos_cp = pltpu.async_copy(pos_hbm.at[pl.ds(off, _CHUNK)], pos_v, sem_p)
        gathers = []
        for b in range(B):
            idx_cps[b].wait()
            gathers.append(
                pltpu.async_copy(tok_hbm.at[idx_v.at[b]], tok_v.at[b], sem_g)
            )
        stores = []
        for b in range(B):
            gathers[b].wait()
            if b == 0:
                pos_cp.wait()

            @plsc.parallel_loop(0, _CHUNK, step=1, unroll=4)
            def add_row(r):
                for c in range(D // _LANES):
                    v = pos_v[r, pl.ds(c * _LANES, _LANES)]
                    plsc.addupdate(tok_v.at[b, r, pl.ds(c * _LANES, _LANES)], v)

            stores.append(
                pltpu.async_copy(
                    tok_v.at[b], out_hbm.at[pl.ds(b * S + off, _CHUNK)], sem_s
                )
            )
        for b in range(B):
            stores[b].wait()

    return emb


def kernel(input_ids, token_weight, pos_weight):
    B, S = input_ids.shape
    V, D = token_weight.shape
    info = plsc.get_sparse_core_info()
    NC, NS = info.num_cores, info.num_subcores
    emb = _build(B, S, V, D, NC, NS)
    out = emb(input_ids.astype(jnp.int32), token_weight, pos_weight)
    return out.reshape(B, S, D)


# speedup vs baseline: 1.0439x; 1.0042x over previous
import functools

import jax
import jax.numpy as jnp
from jax import lax
from jax.experimental import pallas as pl
from jax.experimental.pallas import tpu as pltpu
from jax.experimental.pallas import tpu_sc as plsc

_LANES = 16
_CHUNK = 128


def _build(B, S, V, D, NC, NS):
    N = B * S
    mesh = plsc.VectorSubcoreMesh(core_axis_name="c", subcore_axis_name="s")

    @functools.partial(
        pl.kernel,
        mesh=mesh,
        out_type=jax.ShapeDtypeStruct((N, D), jnp.float32),
        scratch_types=[
            pltpu.VMEM((B, _CHUNK), jnp.int32),
            pltpu.VMEM((_CHUNK, D), jnp.float32),
            pltpu.VMEM((B, _CHUNK, D), jnp.float32),
            pltpu.SemaphoreType.DMA,
            pltpu.SemaphoreType.DMA,
            pltpu.SemaphoreType.DMA,
            pltpu.SemaphoreType.DMA,
        ],
    )
    def emb(ids_hbm, tok_hbm, pos_hbm, out_hbm, idx_v, pos_v, tok_v, sem_p, sem_i, sem_g, sem_s):
        wid = lax.axis_index("s") * NC + lax.axis_index("c")
        off = wid * _CHUNK
        idx_cp = pltpu.async_copy(ids_hbm.at[:, pl.ds(off, _CHUNK)], idx_v, sem_i)
        pos_cp = pltpu.async_copy(pos_hbm.at[pl.ds(off, _CHUNK)], pos_v, sem_p)
        idx_cp.wait()
        gathers = []
        for b in range(B):
            gathers.append(
                pltpu.async_copy(tok_hbm.at[idx_v.at[b]], tok_v.at[b], sem_g)
            )
        stores = []
        for b in range(B):
            gathers[b].wait()
            if b == 0:
                pos_cp.wait()

            @plsc.parallel_loop(0, _CHUNK, step=1, unroll=4)
            def add_row(r):
                for c in range(D // _LANES):
                    v = pos_v[r, pl.ds(c * _LANES, _LANES)]
                    plsc.addupdate(tok_v.at[b, r, pl.ds(c * _LANES, _LANES)], v)

            stores.append(
                pltpu.async_copy(
                    tok_v.at[b], out_hbm.at[pl.ds(b * S + off, _CHUNK)], sem_s
                )
            )
        for b in range(B):
            stores[b].wait()

    return emb


def kernel(input_ids, token_weight, pos_weight):
    B, S = input_ids.shape
    V, D = token_weight.shape
    info = plsc.get_sparse_core_info()
    NC, NS = info.num_cores, info.num_subcores
    emb = _build(B, S, V, D, NC, NS)
    out = emb(input_ids.astype(jnp.int32), token_weight, pos_weight)
    return out.reshape(B, S, D)
